# baseline (speedup 1.0000x reference)
import functools

import jax
import jax.numpy as jnp
from jax import lax
from jax.experimental import pallas as pl
from jax.experimental.pallas import tpu as pltpu
from jax.experimental.pallas import tpu_sc as plsc

_LANES = 16
_NC = 2
_NS = 16
_CH = 16
_PAIR = 2
_RING = 4


def kernel(input_ids, wte, wpe):
    b, s = input_ids.shape
    _, e = wte.shape
    n = b * s
    ids_flat = input_ids.reshape(n).astype(jnp.int32)

    nunits = _NC * _NS
    ppu = s // nunits
    nchunks = ppu // _CH
    npairs = b // _PAIR
    ngroups = npairs * nchunks
    nbufs = _RING * _PAIR

    mesh = plsc.VectorSubcoreMesh(core_axis_name="c", subcore_axis_name="s")

    scratch = (
        [pltpu.VMEM((b * ppu,), jnp.int32)]
        + [pltpu.VMEM((_CH, e), jnp.float32) for _ in range(2)]
        + [pltpu.VMEM((_CH, e), jnp.float32) for _ in range(nbufs)]
        + [pltpu.SemaphoreType.DMA for _ in range(2 + b + 2 * nbufs)]
    )

    @functools.partial(
        pl.kernel,
        out_type=jax.ShapeDtypeStruct((n, e), jnp.float32),
        mesh=mesh,
        scratch_types=scratch,
    )
    def run(wte_hbm, ids_hbm, wpe_hbm, out_hbm, ids_v, wpe0, wpe1, *rest):
        wpe_bufs = (wpe0, wpe1)
        rows = rest[:nbufs]
        sem_wpe = rest[nbufs:nbufs + 2]
        sem_ids = rest[nbufs + 2:nbufs + 2 + b]
        sem_g = rest[nbufs + 2 + b:nbufs + 2 + b + nbufs]
        sem_o = rest[nbufs + 2 + b + nbufs:]

        wid = lax.axis_index("s") * _NC + lax.axis_index("c")
        pos0 = wid * ppu

        h_ids = [
            pltpu.async_copy(
                ids_hbm.at[pl.ds(bb * s + pos0, ppu)],
                ids_v.at[pl.ds(bb * ppu, ppu)],
                sem_ids[bb],
            )
            for bb in range(b)
        ]
        ids_ready = [False] * b

        def fire_wpe(c):
            return pltpu.async_copy(
                wpe_hbm.at[pl.ds(pos0 + c * _CH, _CH)],
                wpe_bufs[c % 2],
                sem_wpe[c % 2],
            )

        def bufs_of(g):
            k = g % _RING
            return rows[k * _PAIR:(k + 1) * _PAIR]

        def group_rows(g):
            c, p = divmod(g, npairs)
            return [p * _PAIR + k for k in range(_PAIR)], c

        def fire_gathers(g):
            brs, c = group_rows(g)
            hs = []
            for k, bb in enumerate(brs):
                if not ids_ready[bb]:
                    h_ids[bb].wait()
                    ids_ready[bb] = True
                hs.append(pltpu.async_copy(
                    wte_hbm.at[ids_v.at[pl.ds(bb * ppu + c * _CH, _CH)]],
                    bufs_of(g)[k],
                    sem_g[(g % _RING) * _PAIR + k],
                ))
            return hs

        hw = {0: fire_wpe(0), 1: fire_wpe(1)}
        wpe_waited = [False, False]

        hg, ho = {}, {}
        for g in range(_RING - 1):
            hg[g] = fire_gathers(g)

        for g in range(ngroups):
            brs, c = group_rows(g)
            gbufs = bufs_of(g)
            with jax.named_scope("gwait"):
                for h in hg[g]:
                    h.wait()
            if not wpe_waited[c % 2]:
                hw[c].wait()
                wpe_waited[c % 2] = True
            wv = wpe_bufs[c % 2]

            with jax.named_scope("add"):
                @plsc.parallel_loop(0, _CH)
                def _(r):
                    for cc in range(0, e, _LANES):
                        slc = (pl.ds(r, 1), pl.ds(cc, _LANES))
                        w = wv.at[slc][...]
                        for gb in gbufs:
                            plsc.addupdate(gb.at[slc], w)

            ho[g] = [
                pltpu.async_copy(
                    gbufs[k],
                    out_hbm.at[pl.ds(bb * s + pos0 + c * _CH, _CH)],
                    sem_o[(g % _RING) * _PAIR + k],
                )
                for k, bb in enumerate(brs)
            ]

            if (g + 1) % npairs == 0 and c + 2 < nchunks:
                hw[c + 2] = fire_wpe(c + 2)
                wpe_waited[c % 2] = False

            nxt = g + _RING - 1
            if nxt < ngroups:
                if g >= 1:
                    with jax.named_scope("owait"):
                        for h in ho[g - 1]:
                            h.wait()
                hg[nxt] = fire_gathers(nxt)

        for g in range(max(0, ngroups - _RING), ngroups):
            if g in ho:
                for h in ho[g]:
                    h.wait()

    out = run(wte, ids_flat, wpe)
    return out.reshape(b, s, e)


# speedup vs baseline: 1.0073x; 1.0073x over previous
import functools

import jax
import jax.numpy as jnp
from jax import lax
from jax.experimental import pallas as pl
from jax.experimental.pallas import tpu as pltpu
from jax.experimental.pallas import tpu_sc as plsc

_LANES = 16
_NC = 2
_NS = 16
_CH = 16
_PAIR = 2
_RING = 4


def kernel(input_ids, wte, wpe):
    b, s = input_ids.shape
    _, e = wte.shape
    n = b * s
    ids_2d = input_ids.astype(jnp.int32)

    nunits = _NC * _NS
    ppu = s // nunits
    nchunks = ppu // _CH
    npairs = b // _PAIR
    ngroups = npairs * nchunks
    nbufs = _RING * _PAIR

    mesh = plsc.VectorSubcoreMesh(core_axis_name="c", subcore_axis_name="s")

    scratch = (
        [pltpu.VMEM((b * ppu,), jnp.int32)]
        + [pltpu.VMEM((_CH, e), jnp.float32) for _ in range(2)]
        + [pltpu.VMEM((_CH, e), jnp.float32) for _ in range(nbufs)]
        + [pltpu.SemaphoreType.DMA for _ in range(2 + b + 2 * nbufs)]
    )

    @functools.partial(
        pl.kernel,
        out_type=jax.ShapeDtypeStruct((b, s, e), jnp.float32),
        mesh=mesh,
        scratch_types=scratch,
    )
    def run(wte_hbm, ids_hbm, wpe_hbm, out_hbm, ids_v, wpe0, wpe1, *rest):
        wpe_bufs = (wpe0, wpe1)
        rows = rest[:nbufs]
        sem_wpe = rest[nbufs:nbufs + 2]
        sem_ids = rest[nbufs + 2:nbufs + 2 + b]
        sem_g = rest[nbufs + 2 + b:nbufs + 2 + b + nbufs]
        sem_o = rest[nbufs + 2 + b + nbufs:]

        wid = lax.axis_index("s") * _NC + lax.axis_index("c")
        pos0 = wid * ppu

        h_ids = [
            pltpu.async_copy(
                ids_hbm.at[bb, pl.ds(pos0, ppu)],
                ids_v.at[pl.ds(bb * ppu, ppu)],
                sem_ids[bb],
            )
            for bb in range(b)
        ]
        ids_ready = [False] * b

        def fire_wpe(c):
            return pltpu.async_copy(
                wpe_hbm.at[pl.ds(pos0 + c * _CH, _CH)],
                wpe_bufs[c % 2],
                sem_wpe[c % 2],
            )

        def bufs_of(g):
            k = g % _RING
            return rows[k * _PAIR:(k + 1) * _PAIR]

        def group_rows(g):
            c, p = divmod(g, npairs)
            return [p * _PAIR + k for k in range(_PAIR)], c

        def fire_gathers(g):
            brs, c = group_rows(g)
            hs = []
            for k, bb in enumerate(brs):
                if not ids_ready[bb]:
                    h_ids[bb].wait()
                    ids_ready[bb] = True
                hs.append(pltpu.async_copy(
                    wte_hbm.at[ids_v.at[pl.ds(bb * ppu + c * _CH, _CH)]],
                    bufs_of(g)[k],
                    sem_g[(g % _RING) * _PAIR + k],
                ))
            return hs

        hw = {0: fire_wpe(0), 1: fire_wpe(1)}
        wpe_waited = [False, False]

        hg, ho = {}, {}
        for g in range(_RING - 1):
            hg[g] = fire_gathers(g)

        for g in range(ngroups):
            brs, c = group_rows(g)
            gbufs = bufs_of(g)
            with jax.named_scope("gwait"):
                for h in hg[g]:
                    h.wait()
            if not wpe_waited[c % 2]:
                hw[c].wait()
                wpe_waited[c % 2] = True
            wv = wpe_bufs[c % 2]

            with jax.named_scope("add"):
                @plsc.parallel_loop(0, _CH)
                def _(r):
                    for cc in range(0, e, _LANES):
                        slc = (pl.ds(r, 1), pl.ds(cc, _LANES))
                        w = wv.at[slc][...]
                        for gb in gbufs:
                            plsc.addupdate(gb.at[slc], w)

            ho[g] = [
                pltpu.async_copy(
                    gbufs[k],
                    out_hbm.at[bb, pl.ds(pos0 + c * _CH, _CH)],
                    sem_o[(g % _RING) * _PAIR + k],
                )
                for k, bb in enumerate(brs)
            ]

            if (g + 1) % npairs == 0 and c + 2 < nchunks:
                hw[c + 2] = fire_wpe(c + 2)
                wpe_waited[c % 2] = False

            nxt = g + _RING - 1
            if nxt < ngroups:
                if g >= 1:
                    with jax.named_scope("owait"):
                        for h in ho[g - 1]:
                            h.wait()
                hg[nxt] = fire_gathers(nxt)

        for g in range(max(0, ngroups - _RING), ngroups):
            if g in ho:
                for h in ho[g]:
                    h.wait()

    return run(wte, ids_2d, wpe)
